# baseline (speedup 1.0000x reference)
import functools

import jax
import jax.numpy as jnp
from jax import lax
from jax.experimental import pallas as pl
from jax.experimental.pallas import tpu as pltpu
from jax.experimental.pallas import tpu_sc as plsc

BLOCK = 128
S = 5
L = 3


def _body(bpw, nc, idx_hbm, table_hbm, out_hbm, idx_all, rows_v, *sems):
    wid = lax.axis_index("s") * nc + lax.axis_index("c")
    blk0 = wid * bpw
    gsems, wsems = sems[:S], sems[S:]

    pltpu.sync_copy(idx_hbm.at[pl.ds(blk0, bpw)], idx_all)

    def fire_g(c, s):
        pltpu.async_copy(table_hbm.at[idx_all.at[c]], rows_v.at[s], gsems[s])

    def wait_g(c, s):
        pltpu.make_async_copy(table_hbm.at[idx_all.at[c]],
                              rows_v.at[s], gsems[s]).wait()

    def fire_w(c, s):
        pltpu.async_copy(rows_v.at[s], out_hbm.at[blk0 + c], wsems[s])

    def wait_w(s):
        pltpu.make_async_copy(rows_v.at[s], out_hbm.at[blk0], wsems[s]).wait()

    def stage(c, s, do_wait_w=True, fire_next=True):
        wait_g(c, s)
        fire_w(c, s)
        if do_wait_w or fire_next:
            t = (s + L) % S
        if do_wait_w:
            wait_w(t)
        if fire_next:
            fire_g(c + L, t)

    for c in range(L):
        fire_g(c, c)
    for c in range(S - L):
        stage(c, c, do_wait_w=False)
    for c in range(S - L, S):
        stage(c, c)

    @pl.loop(1, bpw // S - 1)
    def _main(k):
        c0 = S * k
        for s in range(S):
            stage(c0 + s, s)

    for c in range(bpw - S, bpw):
        stage(c, c % S, fire_next=(c + L < bpw))
    for c in range(bpw + L - S, bpw):
        wait_w(c % S)


def kernel(token_ids, weights):
    b, t = token_ids.shape
    vocab, d = weights.shape
    nb = (b * t) // BLOCK
    info = plsc.get_sparse_core_info()
    nw = info.num_cores * info.num_subcores
    bpw = nb // nw

    idx2d = token_ids.reshape(nb, BLOCK).astype(jnp.int32)
    mesh = plsc.VectorSubcoreMesh(core_axis_name="c", subcore_axis_name="s")
    run = pl.kernel(
        functools.partial(_body, bpw, info.num_cores),
        out_type=jax.ShapeDtypeStruct((nb, BLOCK, d), jnp.float32),
        mesh=mesh,
        scratch_types=[
            pltpu.VMEM((bpw, BLOCK), jnp.int32),
            pltpu.VMEM((S, BLOCK, d), jnp.float32),
        ] + [pltpu.SemaphoreType.DMA] * (2 * S),
    )
    out = run(idx2d, weights)
    return out.reshape(b, t, d)
